# baseline (speedup 1.0000x reference)
import functools

import jax
import jax.numpy as jnp
from jax import lax
from jax.experimental import pallas as pl
from jax.experimental.pallas import tpu as pltpu
from jax.experimental.pallas import tpu_sc as plsc

N = 10000
E = 320000
NC = 2
NS = 16
NW = NC * NS
EPW = E // NW
CH = 128
NCHUNK = 80
EPAD = NW * NCHUNK * CH
NBUF = 4
NPAD = 10240
RPT = NPAD // NS
CNTW = 8


def _seg_sum_sc(table, edge3):
    C = table.shape[1]
    NB1 = 8
    mesh = plsc.VectorSubcoreMesh(core_axis_name="c", subcore_axis_name="s")
    out_type = jax.ShapeDtypeStruct((NC, NPAD, C), jnp.float32)
    scratch = [
        pltpu.VMEM((NCHUNK, CH), jnp.int32),
        pltpu.VMEM((NCHUNK, CH), jnp.int32),
        [pltpu.VMEM((CH, C), jnp.float32) for _ in range(NB1)],
        [pltpu.SemaphoreType.DMA for _ in range(NB1)],
        pltpu.VMEM_SHARED((NPAD, C), jnp.float32),
    ]
    zeros_c = jnp.zeros((NPAD, C), jnp.float32)

    def body(tbl, e_h, zc_h, acc_o, sidx, didx, rows, sems, acc_sh):
        c = lax.axis_index("c")
        s = lax.axis_index("s")
        wid = c * NS + s
        r0 = s * RPT
        pltpu.sync_copy(zc_h.at[pl.ds(r0, RPT)], acc_sh.at[pl.ds(r0, RPT)])
        pltpu.sync_copy(e_h.at[0, wid], sidx)
        pltpu.sync_copy(e_h.at[1, wid], didx)
        plsc.subcore_barrier()

        @pl.loop(0, NCHUNK, step=NB1)
        def group(g):
            ds = [pltpu.async_copy(tbl.at[sidx.at[g + b]], rows[b], sems[b])
                  for b in range(NB1)]
            for b in range(NB1):
                ds[b].wait()
                pltpu.sync_copy(rows[b], acc_sh.at[didx.at[g + b]], add=True)

        plsc.subcore_barrier()
        pltpu.sync_copy(acc_sh.at[pl.ds(r0, RPT)], acc_o.at[c, pl.ds(r0, RPT)])

    k = pl.kernel(body, out_type=out_type, mesh=mesh,
                  scratch_types=scratch,
                  compiler_params=pltpu.CompilerParams(
                      use_tc_tiling_on_sc=False))
    return k(table, edge3, zeros_c)


def _tc_pre(x, W3l, W3r):
    BN = 1280
    D = x.shape[1]
    H = W3l.shape[1]

    def body(x_ref, wl_ref, wr_ref, p_ref, xr_ref):
        xb = x_ref[...]
        pb = jnp.dot(xb, wl_ref[...], preferred_element_type=jnp.float32)
        p_ref[...] = jnp.concatenate(
            [pb, jnp.ones((BN, CNTW), jnp.float32)], axis=1)
        xr_ref[...] = jnp.dot(xb, wr_ref[...], preferred_element_type=jnp.float32)

    return pl.pallas_call(
        body,
        grid=(NPAD // BN,),
        in_specs=[
            pl.BlockSpec((BN, D), lambda i: (i, 0)),
            pl.BlockSpec((D, H), lambda i: (0, 0)),
            pl.BlockSpec((D, H), lambda i: (0, 0)),
        ],
        out_specs=[pl.BlockSpec((BN, H + CNTW), lambda i: (i, 0)),
                   pl.BlockSpec((BN, H), lambda i: (i, 0))],
        out_shape=[jax.ShapeDtypeStruct((NPAD, H + CNTW), jnp.float32),
                   jax.ShapeDtypeStruct((NPAD, H), jnp.float32)],
    )(x, W3l, W3r)


def _sc_layer4_fused(acc1, xrp, b3, edge3):
    C = 32
    CW = acc1.shape[2]
    mesh = plsc.VectorSubcoreMesh(core_axis_name="c", subcore_axis_name="s")
    out_type = [jax.ShapeDtypeStruct((NPAD, C), jnp.float32),
                jax.ShapeDtypeStruct((NC, NPAD, C), jnp.float32)]
    scratch = [
        pltpu.VMEM((NCHUNK, CH), jnp.int32),
        pltpu.VMEM((NCHUNK, CH), jnp.int32),
        [pltpu.VMEM((CH, C), jnp.float32) for _ in range(NBUF)],
        [pltpu.SemaphoreType.DMA for _ in range(NBUF)],
        pltpu.VMEM_SHARED((NPAD, C), jnp.float32),
        pltpu.VMEM((RPT, CW), jnp.float32),
        pltpu.VMEM((RPT, CW), jnp.float32),
        pltpu.VMEM((RPT, C), jnp.float32),
        pltpu.VMEM((RPT,), jnp.float32),
        pltpu.VMEM((C,), jnp.float32),
    ]
    zeros_c = jnp.zeros((NPAD, C), jnp.float32)

    def body(a1_h, xr_h, b3_h, e_h, zc_h, h_o, acc_o,
             sidx, didx, rows, sems, acc_sh, a0v, a1v, xrv, rdv, b3v):
        c = lax.axis_index("c")
        s = lax.axis_index("s")
        wid = c * NS + s
        r0 = s * RPT
        pre = [
            pltpu.async_copy(zc_h.at[pl.ds(r0, RPT)],
                             acc_sh.at[pl.ds(r0, RPT)], sems[0]),
            pltpu.async_copy(e_h.at[0, wid], sidx, sems[1]),
            pltpu.async_copy(e_h.at[1, wid], didx, sems[2]),
            pltpu.async_copy(a1_h.at[0, pl.ds(r0, RPT)], a0v, sems[3]),
        ]
        pltpu.sync_copy(a1_h.at[1, pl.ds(r0, RPT)], a1v)
        pltpu.sync_copy(xr_h.at[pl.ds(r0, RPT)], xrv)
        pltpu.sync_copy(b3_h, b3v)
        for d in pre:
            d.wait()

        @pl.loop(0, RPT, step=16)
        def deg16(g):
            ridx = g + lax.iota(jnp.int32, 16)
            c32 = jnp.full((16,), C, jnp.int32)
            d0 = plsc.load_gather(a0v, [ridx, c32])
            d1 = plsc.load_gather(a1v, [ridx, c32])
            rdv[pl.ds(g, 16)] = 1.0 / jnp.maximum(d0 + d1, 1.0)

        @pl.loop(0, RPT, unroll=4)
        def hrow(r):
            rd = plsc.load_gather(rdv, [jnp.full((16,), r, jnp.int32)])
            for half in range(2):
                cs = pl.ds(half * 16, 16)
                v = ((a0v[r, cs] + a1v[r, cs]) * rd + b3v[cs] + xrv[r, cs])
                xrv[r, cs] = jnp.maximum(v, 0.0)

        pltpu.sync_copy(xrv, h_o.at[pl.ds(r0, RPT)])
        plsc.subcore_barrier()

        @pl.loop(0, NCHUNK, step=NBUF)
        def group(g):
            ds = [pltpu.async_copy(h_o.at[sidx.at[g + b]], rows[b], sems[b])
                  for b in range(NBUF)]
            for b in range(NBUF):
                ds[b].wait()
                pltpu.sync_copy(rows[b], acc_sh.at[didx.at[g + b]], add=True)

        plsc.subcore_barrier()
        pltpu.sync_copy(acc_sh.at[pl.ds(r0, RPT)], acc_o.at[c, pl.ds(r0, RPT)])

    k = pl.kernel(body, out_type=out_type, mesh=mesh, scratch_types=scratch,
                  compiler_params=pltpu.CompilerParams(
                      use_tc_tiling_on_sc=False, needs_layout_passes=False))
    return k(acc1, xrp, b3, edge3, zeros_c)


def _tc_out(acc2, acc1, h, W4l, W4r, b4):
    BN = 2000
    H = h.shape[1]
    CW = acc1.shape[2]
    O = W4l.shape[1]

    def body(a_ref, a1_ref, h_ref, wl_ref, wr_ref, b_ref, o_ref):
        a = a_ref[...]
        a1 = a1_ref[...]
        deg = a1[0, :, H:H + 1] + a1[1, :, H:H + 1]
        mean = (a[0] + a[1]) / jnp.maximum(deg, 1.0)
        o = (jnp.dot(mean, wl_ref[...], preferred_element_type=jnp.float32)
             + b_ref[...]
             + jnp.dot(h_ref[...], wr_ref[...],
                       preferred_element_type=jnp.float32))
        m = jnp.max(o, axis=1, keepdims=True)
        eo = jnp.exp(o - m)
        o_ref[...] = o - m - jnp.log(jnp.sum(eo, axis=1, keepdims=True))

    return pl.pallas_call(
        body,
        grid=(N // BN,),
        in_specs=[
            pl.BlockSpec((NC, BN, H), lambda i: (0, i, 0)),
            pl.BlockSpec((NC, BN, CW), lambda i: (0, i, 0)),
            pl.BlockSpec((BN, H), lambda i: (i, 0)),
            pl.BlockSpec((H, O), lambda i: (0, 0)),
            pl.BlockSpec((H, O), lambda i: (0, 0)),
            pl.BlockSpec((1, O), lambda i: (0, 0)),
        ],
        out_specs=pl.BlockSpec((BN, O), lambda i: (i, 0)),
        out_shape=jax.ShapeDtypeStruct((N, O), jnp.float32),
    )(acc2, acc1, h, W4l, W4r, b4)


def kernel(x, edge_index, W1l, b1, W1r, W2l, b2, W2r, W3l, b3, W3r,
           W4l, b4, W4r):
    pad = EPAD - E
    srcp = jnp.concatenate([edge_index[0], jnp.zeros((pad,), jnp.int32)])
    dstp = jnp.concatenate([edge_index[1],
                            N + jnp.arange(pad, dtype=jnp.int32) % (NPAD - N)])
    edge3 = jnp.stack([srcp, dstp]).reshape(2, NW, NCHUNK, CH)
    p, xr = _tc_pre(x, W3l, W3r)
    acc1 = _seg_sum_sc(p, edge3)
    h, acc2 = _sc_layer4_fused(acc1, xr, b3, edge3)
    return _tc_out(acc2, acc1, h, W4l, W4r, b4.reshape(1, -1))


# speedup vs baseline: 2.0684x; 2.0684x over previous
import functools

import jax
import jax.numpy as jnp
from jax import lax
from jax.experimental import pallas as pl
from jax.experimental.pallas import tpu as pltpu
from jax.experimental.pallas import tpu_sc as plsc

N = 10000
E = 320000
NC = 2
NS = 16
NW = NC * NS
EPW = E // NW
CH = 125
NCHUNK = EPW // CH
NBUF = 4
NPAD = 10240
RPT = NPAD // NS
CNTW = 8


def _seg_sum_sc(table, edge3):
    C = table.shape[1]
    NB1 = 8
    mesh = plsc.VectorSubcoreMesh(core_axis_name="c", subcore_axis_name="s")
    out_type = jax.ShapeDtypeStruct((NC, NPAD, C), jnp.float32)
    scratch = [
        pltpu.VMEM((NCHUNK, CH), jnp.int32),
        pltpu.VMEM((NCHUNK, CH), jnp.int32),
        [pltpu.VMEM((CH, C), jnp.float32) for _ in range(NB1)],
        [pltpu.SemaphoreType.DMA for _ in range(NB1)],
        pltpu.VMEM_SHARED((NPAD, C), jnp.float32),
    ]
    zeros_c = jnp.zeros((NPAD, C), jnp.float32)

    def body(tbl, e_h, zc_h, acc_o, sidx, didx, rows, sems, acc_sh):
        c = lax.axis_index("c")
        s = lax.axis_index("s")
        wid = c * NS + s
        r0 = s * RPT
        pltpu.sync_copy(zc_h.at[pl.ds(r0, RPT)], acc_sh.at[pl.ds(r0, RPT)])
        pltpu.sync_copy(e_h.at[0, wid], sidx)
        pltpu.sync_copy(e_h.at[1, wid], didx)
        plsc.subcore_barrier()

        @pl.loop(0, NCHUNK, step=NB1)
        def group(g):
            ds = [pltpu.async_copy(tbl.at[sidx.at[g + b]], rows[b], sems[b])
                  for b in range(NB1)]
            for b in range(NB1):
                ds[b].wait()
                pltpu.sync_copy(rows[b], acc_sh.at[didx.at[g + b]], add=True)

        plsc.subcore_barrier()
        pltpu.sync_copy(acc_sh.at[pl.ds(r0, RPT)], acc_o.at[c, pl.ds(r0, RPT)])

    k = pl.kernel(body, out_type=out_type, mesh=mesh,
                  scratch_types=scratch,
                  compiler_params=pltpu.CompilerParams(
                      use_tc_tiling_on_sc=False))
    return k(table, edge3, zeros_c)


def _tc_pre(x, W3l, W3r):
    BN = 1280
    D = x.shape[1]
    H = W3l.shape[1]

    def body(x_ref, wl_ref, wr_ref, p_ref, xr_ref):
        xb = x_ref[...]
        pb = jnp.dot(xb, wl_ref[...], preferred_element_type=jnp.float32)
        p_ref[...] = jnp.concatenate(
            [pb, jnp.ones((BN, CNTW), jnp.float32)], axis=1)
        xr_ref[...] = jnp.dot(xb, wr_ref[...], preferred_element_type=jnp.float32)

    return pl.pallas_call(
        body,
        grid=(NPAD // BN,),
        in_specs=[
            pl.BlockSpec((BN, D), lambda i: (i, 0)),
            pl.BlockSpec((D, H), lambda i: (0, 0)),
            pl.BlockSpec((D, H), lambda i: (0, 0)),
        ],
        out_specs=[pl.BlockSpec((BN, H + CNTW), lambda i: (i, 0)),
                   pl.BlockSpec((BN, H), lambda i: (i, 0))],
        out_shape=[jax.ShapeDtypeStruct((NPAD, H + CNTW), jnp.float32),
                   jax.ShapeDtypeStruct((NPAD, H), jnp.float32)],
    )(x, W3l, W3r)


def _sc_layer4_fused(acc1, xrp, b3, edge3):
    C = 32
    CW = acc1.shape[2]
    mesh = plsc.VectorSubcoreMesh(core_axis_name="c", subcore_axis_name="s")
    out_type = [jax.ShapeDtypeStruct((NPAD, C), jnp.float32),
                jax.ShapeDtypeStruct((NC, NPAD, C), jnp.float32)]
    scratch = [
        pltpu.VMEM((NCHUNK, CH), jnp.int32),
        pltpu.VMEM((NCHUNK, CH), jnp.int32),
        [pltpu.VMEM((CH, C), jnp.float32) for _ in range(NBUF)],
        [pltpu.SemaphoreType.DMA for _ in range(NBUF)],
        pltpu.VMEM_SHARED((NPAD, C), jnp.float32),
        pltpu.VMEM((RPT, CW), jnp.float32),
        pltpu.VMEM((RPT, CW), jnp.float32),
        pltpu.VMEM((RPT, C), jnp.float32),
        pltpu.VMEM((RPT,), jnp.float32),
        pltpu.VMEM((C,), jnp.float32),
    ]
    zeros_c = jnp.zeros((NPAD, C), jnp.float32)

    def body(a1_h, xr_h, b3_h, e_h, zc_h, h_o, acc_o,
             sidx, didx, rows, sems, acc_sh, a0v, a1v, xrv, rdv, b3v):
        c = lax.axis_index("c")
        s = lax.axis_index("s")
        wid = c * NS + s
        r0 = s * RPT
        pre = [
            pltpu.async_copy(zc_h.at[pl.ds(r0, RPT)],
                             acc_sh.at[pl.ds(r0, RPT)], sems[0]),
            pltpu.async_copy(e_h.at[0, wid], sidx, sems[1]),
            pltpu.async_copy(e_h.at[1, wid], didx, sems[2]),
            pltpu.async_copy(a1_h.at[0, pl.ds(r0, RPT)], a0v, sems[3]),
        ]
        pltpu.sync_copy(a1_h.at[1, pl.ds(r0, RPT)], a1v)
        pltpu.sync_copy(xr_h.at[pl.ds(r0, RPT)], xrv)
        pltpu.sync_copy(b3_h, b3v)
        for d in pre:
            d.wait()

        @pl.loop(0, RPT, step=16)
        def deg16(g):
            ridx = g + lax.iota(jnp.int32, 16)
            c32 = jnp.full((16,), C, jnp.int32)
            d0 = plsc.load_gather(a0v, [ridx, c32])
            d1 = plsc.load_gather(a1v, [ridx, c32])
            rdv[pl.ds(g, 16)] = 1.0 / jnp.maximum(d0 + d1, 1.0)

        @pl.loop(0, RPT, unroll=4)
        def hrow(r):
            rd = plsc.load_gather(rdv, [jnp.full((16,), r, jnp.int32)])
            for half in range(2):
                cs = pl.ds(half * 16, 16)
                v = ((a0v[r, cs] + a1v[r, cs]) * rd + b3v[cs] + xrv[r, cs])
                xrv[r, cs] = jnp.maximum(v, 0.0)

        pltpu.sync_copy(xrv, h_o.at[pl.ds(r0, RPT)])
        plsc.subcore_barrier()

        @pl.loop(0, NCHUNK, step=NBUF)
        def group(g):
            ds = [pltpu.async_copy(h_o.at[sidx.at[g + b]], rows[b], sems[b])
                  for b in range(NBUF)]
            for b in range(NBUF):
                ds[b].wait()
                pltpu.sync_copy(rows[b], acc_sh.at[didx.at[g + b]], add=True)

        plsc.subcore_barrier()
        pltpu.sync_copy(acc_sh.at[pl.ds(r0, RPT)], acc_o.at[c, pl.ds(r0, RPT)])

    k = pl.kernel(body, out_type=out_type, mesh=mesh, scratch_types=scratch,
                  compiler_params=pltpu.CompilerParams(
                      use_tc_tiling_on_sc=False, needs_layout_passes=False))
    return k(acc1, xrp, b3, edge3, zeros_c)


def _tc_out(acc2, acc1, h, W4l, W4r, b4):
    BN = 2000
    H = h.shape[1]
    CW = acc1.shape[2]
    O = W4l.shape[1]

    def body(a_ref, a1_ref, h_ref, wl_ref, wr_ref, b_ref, o_ref):
        a = a_ref[...]
        a1 = a1_ref[...]
        deg = a1[0, :, H:H + 1] + a1[1, :, H:H + 1]
        mean = (a[0] + a[1]) / jnp.maximum(deg, 1.0)
        o = (jnp.dot(mean, wl_ref[...], preferred_element_type=jnp.float32)
             + b_ref[...]
             + jnp.dot(h_ref[...], wr_ref[...],
                       preferred_element_type=jnp.float32))
        m = jnp.max(o, axis=1, keepdims=True)
        eo = jnp.exp(o - m)
        o_ref[...] = o - m - jnp.log(jnp.sum(eo, axis=1, keepdims=True))

    return pl.pallas_call(
        body,
        grid=(N // BN,),
        in_specs=[
            pl.BlockSpec((NC, BN, H), lambda i: (0, i, 0)),
            pl.BlockSpec((NC, BN, CW), lambda i: (0, i, 0)),
            pl.BlockSpec((BN, H), lambda i: (i, 0)),
            pl.BlockSpec((H, O), lambda i: (0, 0)),
            pl.BlockSpec((H, O), lambda i: (0, 0)),
            pl.BlockSpec((1, O), lambda i: (0, 0)),
        ],
        out_specs=pl.BlockSpec((BN, O), lambda i: (i, 0)),
        out_shape=jax.ShapeDtypeStruct((N, O), jnp.float32),
    )(acc2, acc1, h, W4l, W4r, b4)


def kernel(x, edge_index, W1l, b1, W1r, W2l, b2, W2r, W3l, b3, W3r,
           W4l, b4, W4r):
    edge3 = edge_index.reshape(2, NW, NCHUNK, CH)
    p, xr = _tc_pre(x, W3l, W3r)
    acc1 = _seg_sum_sc(p, edge3)
    h, acc2 = _sc_layer4_fused(acc1, xr, b3, edge3)
    return _tc_out(acc2, acc1, h, W4l, W4r, b4.reshape(1, -1))
